# baseline (speedup 1.0000x reference)
import functools

import jax
import jax.numpy as jnp
from jax import lax
from jax.experimental import pallas as pl
from jax.experimental.pallas import tpu as pltpu
from jax.experimental.pallas import tpu_sc as plsc

N = 50000
N_OUT = 100000
P = 12500
C_IN = 128
C_OUT = 64

NC = 2
NS = 16
LANES = 16
CHUNK = 128

BLK = 1000
NB = N // BLK
EPS = 1e-5

IT27 = 166
PAD27 = NS * CHUNK * IT27
IT9 = 56
PAD9 = NS * CHUNK * IT9

DUMMY = 50000
ACC_ROWS = 50176
ZCHUNKS = ACC_ROWS // CHUNK
WCHUNKS = N // CHUNK
WTAIL = N - WCHUNKS * CHUNK


def _cdiv(a, b):
    return (a + b - 1) // b


def _mm_table_call(nj, body, arrs):
    return pl.pallas_call(
        body,
        grid=(NB, nj),
        in_specs=[pl.BlockSpec(s, m) for s, m in arrs],
        out_specs=pl.BlockSpec((BLK, 32), lambda i, j: (j * NB + i, 0)),
        out_shape=jax.ShapeDtypeStruct((nj * N, 32), jnp.float32),
    )


def _k1_body(x_ref, s_ref, w_ref, o_ref):
    x = x_ref[...] + s_ref[...]
    o_ref[...] = jnp.dot(x, w_ref[0], preferred_element_type=jnp.float32)


def _tc_mm1(x, skip, w_j):
    nj = w_j.shape[0]
    arrs = [
        ((BLK, C_IN), lambda i, j: (i, 0)),
        ((BLK, C_IN), lambda i, j: (i, 0)),
        ((1, C_IN, 32), lambda i, j: (j, 0, 0)),
    ]
    return _mm_table_call(nj, _k1_body, arrs)(x, skip, w_j)


def _bn_coeff(s, q, g, b):
    mu = s * (1.0 / N)
    var = q * (1.0 / N) - mu * mu
    sc = g * lax.rsqrt(var + EPS)
    return sc, b - mu * sc


def _k3_body(a0_ref, a1_ref, s_ref, q_ref, g_ref, b_ref, w_ref, o_ref):
    sc, off = _bn_coeff(s_ref[...], q_ref[...], g_ref[...], b_ref[...])
    halves = []
    for h, a in enumerate((a0_ref[...], a1_ref[...])):
        y = jnp.where(a >= 0, a, 0.01 * a)
        halves.append(y * sc[h:h + 1] + off[h:h + 1])
    cat = jnp.concatenate(halves, axis=-1)
    o_ref[...] = jnp.dot(cat, w_ref[0], preferred_element_type=jnp.float32)


def _tc_bn_mm(a_flat, s, q, g2, b2, w_j):
    nj = w_j.shape[0]
    small = ((2, 32), lambda i, j: (0, 0))
    arrs = [
        ((BLK, 32), lambda i, j: (i, 0)),
        ((BLK, 32), lambda i, j: (NB + i, 0)),
        small, small, small, small,
        ((1, C_OUT, 32), lambda i, j: (j, 0, 0)),
    ]
    return _mm_table_call(nj, _k3_body, arrs)(a_flat, a_flat, s, q, g2, b2,
                                              w_j)


def _k5_body(c10_ref, c11_ref, c20_ref, c21_ref, s1_ref, q1_ref, s2_ref,
             q2_ref, g1_ref, b1_ref, g2_ref, b2_ref, w_ref, o_ref):
    sc1, off1 = _bn_coeff(s1_ref[...], q1_ref[...], g1_ref[...], b1_ref[...])
    sc2, off2 = _bn_coeff(s2_ref[...], q2_ref[...], g2_ref[...], b2_ref[...])
    halves = []
    for h, (c1, c2) in enumerate(((c10_ref[...], c20_ref[...]),
                                  (c11_ref[...], c21_ref[...]))):
        halves.append(c1 * sc1[h:h + 1] + off1[h:h + 1]
                      + c2 * sc2[h:h + 1] + off2[h:h + 1])
    e = jnp.concatenate(halves, axis=-1)
    o_ref[...] = jnp.dot(e, w_ref[0], preferred_element_type=jnp.float32)


def _tc_bn2_mm(c1_flat, c2_flat, s1, q1, s2, q2, g1h, b1h, g2h, b2h, w_j):
    nj = w_j.shape[0]
    small = ((2, 32), lambda i, j: (0, 0))
    arrs = [
        ((BLK, 32), lambda i, j: (i, 0)),
        ((BLK, 32), lambda i, j: (NB + i, 0)),
        ((BLK, 32), lambda i, j: (i, 0)),
        ((BLK, 32), lambda i, j: (NB + i, 0)),
        small, small, small, small, small, small, small, small,
        ((1, C_OUT, 32), lambda i, j: (j, 0, 0)),
    ]
    return _mm_table_call(nj, _k5_body, arrs)(
        c1_flat, c1_flat, c2_flat, c2_flat, s1, q1, s2, q2,
        g1h, b1h, g2h, b2h, w_j)


def _k2_body(a0_ref, a1_ref, s_ref, q_ref):
    y0 = a0_ref[...]
    y0 = jnp.where(y0 >= 0, y0, 0.01 * y0)
    y1 = a1_ref[...]
    y1 = jnp.where(y1 >= 0, y1, 0.01 * y1)

    @pl.when(pl.program_id(0) == 0)
    def _():
        s_ref[...] = jnp.zeros_like(s_ref)
        q_ref[...] = jnp.zeros_like(q_ref)

    s_ref[...] += jnp.stack([jnp.sum(y0, 0), jnp.sum(y1, 0)])
    q_ref[...] += jnp.stack([jnp.sum(y0 * y0, 0), jnp.sum(y1 * y1, 0)])


def _tc_stats_leaky(a_flat):
    sspec = pl.BlockSpec((2, 32), lambda i: (0, 0))
    sshape = jax.ShapeDtypeStruct((2, 32), jnp.float32)
    return pl.pallas_call(
        _k2_body,
        grid=(NB,),
        in_specs=[
            pl.BlockSpec((BLK, 32), lambda i: (i, 0)),
            pl.BlockSpec((BLK, 32), lambda i: (NB + i, 0)),
        ],
        out_specs=[sspec, sspec],
        out_shape=[sshape, sshape],
    )(a_flat, a_flat)


def _k4_body(c10_ref, c11_ref, c20_ref, c21_ref, s1_ref, q1_ref, s2_ref,
             q2_ref):
    @pl.when(pl.program_id(0) == 0)
    def _():
        s1_ref[...] = jnp.zeros_like(s1_ref)
        q1_ref[...] = jnp.zeros_like(q1_ref)
        s2_ref[...] = jnp.zeros_like(s2_ref)
        q2_ref[...] = jnp.zeros_like(q2_ref)

    c10 = c10_ref[...]
    c11 = c11_ref[...]
    c20 = c20_ref[...]
    c21 = c21_ref[...]
    s1_ref[...] += jnp.stack([jnp.sum(c10, 0), jnp.sum(c11, 0)])
    q1_ref[...] += jnp.stack([jnp.sum(c10 * c10, 0), jnp.sum(c11 * c11, 0)])
    s2_ref[...] += jnp.stack([jnp.sum(c20, 0), jnp.sum(c21, 0)])
    q2_ref[...] += jnp.stack([jnp.sum(c20 * c20, 0), jnp.sum(c21 * c21, 0)])


def _tc_stats2(c1_flat, c2_flat):
    sspec = pl.BlockSpec((2, 32), lambda i: (0, 0))
    sshape = jax.ShapeDtypeStruct((2, 32), jnp.float32)
    return pl.pallas_call(
        _k4_body,
        grid=(NB,),
        in_specs=[
            pl.BlockSpec((BLK, 32), lambda i: (i, 0)),
            pl.BlockSpec((BLK, 32), lambda i: (NB + i, 0)),
            pl.BlockSpec((BLK, 32), lambda i: (i, 0)),
            pl.BlockSpec((BLK, 32), lambda i: (NB + i, 0)),
        ],
        out_specs=[sspec, sspec, sspec, sspec],
        out_shape=[sshape, sshape, sshape, sshape],
    )(c1_flat, c1_flat, c2_flat, c2_flat)


def _k6_body(a0_ref, a1_ref, o_ref):
    o_ref[...] = jnp.concatenate([a0_ref[...], a1_ref[...]], axis=-1)


def _tc_interleave(a_flat, n_rows):
    nb = n_rows // BLK
    return pl.pallas_call(
        _k6_body,
        grid=(nb,),
        in_specs=[
            pl.BlockSpec((BLK, 32), lambda i: (i, 0)),
            pl.BlockSpec((BLK, 32), lambda i: (nb + i, 0)),
        ],
        out_specs=pl.BlockSpec((BLK, C_OUT), lambda i: (i, 0)),
        out_shape=jax.ShapeDtypeStruct((n_rows, C_OUT), jnp.float32),
    )(a_flat, a_flat)


def _sc_zero_acc(s, zbuf, acc):
    for j in range(_cdiv(ZCHUNKS, NS)):
        ci = s + NS * j

        @pl.when(ci < ZCHUNKS)
        def _():
            pltpu.sync_copy(zbuf, acc.at[pl.ds(ci * CHUNK, CHUNK)])


def _sc_writeback(s, acc, out_hbm, out_base):
    for j in range(_cdiv(WCHUNKS, NS)):
        ci = s + NS * j

        @pl.when(ci < WCHUNKS)
        def _():
            pltpu.sync_copy(acc.at[pl.ds(ci * CHUNK, CHUNK)],
                            out_hbm.at[pl.ds(out_base + ci * CHUNK, CHUNK)])

    @pl.when(s == 0)
    def _():
        pltpu.sync_copy(acc.at[pl.ds(WCHUNKS * CHUNK, WTAIL)],
                        out_hbm.at[pl.ds(out_base + WCHUNKS * CHUNK, WTAIL)])


def _sc_pair_loop(s, tab, g_hbm, s_hbm, goff, soff, iters, acc,
                  idxg, idxs, rows, semg):
    base0 = s * (iters * CHUNK)
    half = iters // 2

    def fire(b, i):
        base = base0 + i * CHUNK
        pltpu.sync_copy(g_hbm.at[pl.ds(goff + base, CHUNK)], idxg[b])
        pltpu.sync_copy(s_hbm.at[pl.ds(soff + base, CHUNK)], idxs[b])
        pltpu.make_async_copy(tab.at[idxg[b]], rows[b], semg[b]).start()

    fire(0, 0)
    fire(1, 1)

    def body(t, _):
        for b in range(2):
            pltpu.make_async_copy(tab.at[idxg[b]], rows[b], semg[b]).wait()
            pltpu.sync_copy(rows[b], acc.at[idxs[b]], add=True)

            @pl.when(t < half - 1)
            def _():
                fire(b, 2 * t + 2 + b)

        return 0

    lax.fori_loop(0, half, body, 0)


def _sc_mesh():
    return plsc.VectorSubcoreMesh(core_axis_name="c", subcore_axis_name="s")


def _sc_scratch():
    return [
        pltpu.VMEM((CHUNK,), jnp.int32),
        pltpu.VMEM((CHUNK,), jnp.int32),
        pltpu.VMEM((CHUNK,), jnp.int32),
        pltpu.VMEM((CHUNK,), jnp.int32),
        pltpu.VMEM((CHUNK, 32), jnp.float32),
        pltpu.VMEM((CHUNK, 32), jnp.float32),
        pltpu.VMEM((CHUNK, 32), jnp.float32),
        pltpu.VMEM_SHARED((ACC_ROWS, 32), jnp.float32),
        pltpu.SemaphoreType.DMA,
        pltpu.SemaphoreType.DMA,
    ]


def _make_sc_conv():
    @functools.partial(
        pl.kernel,
        out_type=jax.ShapeDtypeStruct((2 * N, 32), jnp.float32),
        mesh=_sc_mesh(),
        compiler_params=pltpu.CompilerParams(use_tc_tiling_on_sc=False),
        scratch_types=_sc_scratch(),
    )
    def k(tab, g_hbm, s_hbm, zrows, out_hbm,
          idxg0, idxg1, idxs0, idxs1, rows0, rows1, zbuf, acc, sem0, sem1):
        c = lax.axis_index("c")
        s = lax.axis_index("s")
        pltpu.sync_copy(zrows, zbuf)
        _sc_zero_acc(s, zbuf, acc)
        plsc.subcore_barrier()
        _sc_pair_loop(s, tab, g_hbm, s_hbm, c * PAD27, 0, IT27, acc,
                      (idxg0, idxg1), (idxs0, idxs1), (rows0, rows1),
                      (sem0, sem1))
        plsc.subcore_barrier()
        _sc_writeback(s, acc, out_hbm, c * N)

    return k


def _make_sc_conv2():
    @functools.partial(
        pl.kernel,
        out_type=(jax.ShapeDtypeStruct((2 * N, 32), jnp.float32),
                  jax.ShapeDtypeStruct((2 * N, 32), jnp.float32)),
        mesh=_sc_mesh(),
        compiler_params=pltpu.CompilerParams(use_tc_tiling_on_sc=False),
        scratch_types=_sc_scratch(),
    )
    def k(tab, g1_hbm, s1_hbm, g2_hbm, s2_hbm, zrows, out1, out2,
          idxg0, idxg1, idxs0, idxs1, rows0, rows1, zbuf, acc, sem0, sem1):
        c = lax.axis_index("c")
        s = lax.axis_index("s")
        pltpu.sync_copy(zrows, zbuf)
        for gi, si, oi in ((g1_hbm, s1_hbm, out1), (g2_hbm, s2_hbm, out2)):
            _sc_zero_acc(s, zbuf, acc)
            plsc.subcore_barrier()
            _sc_pair_loop(s, tab, gi, si, c * PAD9, 0, IT9, acc,
                          (idxg0, idxg1), (idxs0, idxs1), (rows0, rows1),
                          (sem0, sem1))
            plsc.subcore_barrier()
            _sc_writeback(s, acc, oi, c * N)
            plsc.subcore_barrier()

    return k


def _make_sc_up():
    @functools.partial(
        pl.kernel,
        out_type=jax.ShapeDtypeStruct((2 * N_OUT, 32), jnp.float32),
        mesh=_sc_mesh(),
        compiler_params=pltpu.CompilerParams(use_tc_tiling_on_sc=False),
        scratch_types=_sc_scratch(),
    )
    def k(tab, g_hbm, s_hbm, zrows, out_hbm,
          idxg0, idxg1, idxs0, idxs1, rows0, rows1, zbuf, acc, sem0, sem1):
        c = lax.axis_index("c")
        s = lax.axis_index("s")
        pltpu.sync_copy(zrows, zbuf)
        for p in range(2):
            _sc_zero_acc(s, zbuf, acc)
            plsc.subcore_barrier()
            _sc_pair_loop(s, tab, g_hbm, s_hbm, c * PAD27, p * PAD27, IT27,
                          acc, (idxg0, idxg1), (idxs0, idxs1), (rows0, rows1),
                          (sem0, sem1))
            plsc.subcore_barrier()
            _sc_writeback(s, acc, out_hbm, c * N_OUT + p * N)
            plsc.subcore_barrier()

    return k


def _prep_w(w, cin):
    kk = w.shape[0]
    return jnp.transpose(w.reshape(kk, cin, 2, 32), (0, 2, 1, 3)).reshape(
        2 * kk, cin, 32)


def _prep_gidx(rb_in, pad_len, koff=0):
    kk = rb_in.shape[0]
    k_ids = jnp.arange(kk, dtype=jnp.int32)[:, None] + koff
    g = rb_in.astype(jnp.int32) + k_ids * (2 * N)
    g = jnp.pad(g.reshape(-1), (0, pad_len - kk * P))
    return jnp.concatenate([g, g + N])


def _prep_sidx(rb_out, pad_len):
    s = rb_out.astype(jnp.int32).reshape(-1)
    return jnp.pad(s, (0, pad_len - s.shape[0]), constant_values=DUMMY)


def kernel(x_features, skip_features, W_trans, bn_t_g, bn_t_b, W1, bn1_g,
           bn1_b, W2, bn2_g, bn2_b, W_up, rb_trans_in, rb_trans_out, rb1_in,
           rb1_out, rb2_in, rb2_out, rb_up_in, rb_up_out):
    f32 = jnp.float32

    w_t = _prep_w(W_trans, C_IN)
    w_12 = _prep_w(jnp.concatenate([W1, W2], 0), C_OUT)
    w_up = _prep_w(W_up, C_OUT)

    gt = _prep_gidx(rb_trans_in, PAD27)
    st = _prep_sidx(rb_trans_out, PAD27)
    g1 = _prep_gidx(rb1_in, PAD9)
    s1 = _prep_sidx(rb1_out, PAD9)
    g2 = _prep_gidx(rb2_in, PAD9, 9)
    s2 = _prep_sidx(rb2_out, PAD9)
    gu = _prep_gidx(rb_up_in, PAD27)
    su_raw = rb_up_out.astype(jnp.int32).reshape(-1)
    su_raw = jnp.pad(su_raw, (0, PAD27 - 27 * P), constant_values=1 << 29)
    su0 = jnp.where(su_raw < N, su_raw, DUMMY)
    su1 = jnp.where((su_raw >= N) & (su_raw < 2 * N), su_raw - N, DUMMY)
    su = jnp.concatenate([su0, su1])

    zrows = jnp.zeros((CHUNK, 32), f32)

    g2h_t = bn_t_g.reshape(2, 32).astype(f32)
    b2h_t = bn_t_b.reshape(2, 32).astype(f32)
    g2h_1 = bn1_g.reshape(2, 32).astype(f32)
    b2h_1 = bn1_b.reshape(2, 32).astype(f32)
    g2h_2 = bn2_g.reshape(2, 32).astype(f32)
    b2h_2 = bn2_b.reshape(2, 32).astype(f32)

    tab_t = _tc_mm1(x_features, skip_features, w_t)
    a_flat = _make_sc_conv()(tab_t, gt, st, zrows)

    s_t, q_t = _tc_stats_leaky(a_flat)
    tab_12 = _tc_bn_mm(a_flat, s_t, q_t, g2h_t, b2h_t, w_12)

    c1_flat, c2_flat = _make_sc_conv2()(tab_12, g1, s1, g2, s2, zrows)

    s1s, q1s, s2s, q2s = _tc_stats2(c1_flat, c2_flat)
    tab_up = _tc_bn2_mm(c1_flat, c2_flat, s1s, q1s, s2s, q2s,
                        g2h_1, b2h_1, g2h_2, b2h_2, w_up)

    o_flat = _make_sc_up()(tab_up, gu, su, zrows)
    return _tc_interleave(o_flat, N_OUT)


# speedup vs baseline: 3.2293x; 3.2293x over previous
import functools

import jax
import jax.numpy as jnp
from jax import lax
from jax.experimental import pallas as pl
from jax.experimental.pallas import tpu as pltpu
from jax.experimental.pallas import tpu_sc as plsc

N = 50000
N_OUT = 100000
P = 12500
C_IN = 128
C_OUT = 64

NC = 2
NS = 16
LANES = 16
CHUNK = 128

BLK = 1000
NB = N // BLK
EPS = 1e-5

IT27 = 166
PAD27 = NS * CHUNK * IT27
IT9 = 56
PAD9 = NS * CHUNK * IT9

DUMMY = 50000
ACC_ROWS = 50176
ZCHUNKS = ACC_ROWS // CHUNK
WCHUNKS = N // CHUNK
WTAIL = N - WCHUNKS * CHUNK


def _cdiv(a, b):
    return (a + b - 1) // b


def _mm_table_call(nw, body, arrs):
    return pl.pallas_call(
        body,
        grid=(NB,),
        in_specs=[pl.BlockSpec(s, m) for s, m in arrs],
        out_specs=pl.BlockSpec((BLK, nw), lambda i: (i, 0)),
        out_shape=jax.ShapeDtypeStruct((N, nw), jnp.float32),
    )


def _k1_body(x_ref, s_ref, w_ref, o_ref):
    x = x_ref[...] + s_ref[...]
    o_ref[...] = jnp.dot(x, w_ref[...], preferred_element_type=jnp.float32)


def _tc_mm1(x, skip, w_mat):
    nw = w_mat.shape[1]
    arrs = [
        ((BLK, C_IN), lambda i: (i, 0)),
        ((BLK, C_IN), lambda i: (i, 0)),
        ((C_IN, nw), lambda i: (0, 0)),
    ]
    return _mm_table_call(nw, _k1_body, arrs)(x, skip, w_mat)


def _bn_coeff(s, q, g, b):
    mu = s * (1.0 / N)
    var = q * (1.0 / N) - mu * mu
    sc = g * lax.rsqrt(var + EPS)
    return sc, b - mu * sc


def _k3_body(a0_ref, a1_ref, s_ref, q_ref, g_ref, b_ref, w_ref, o_ref):
    sc, off = _bn_coeff(s_ref[...], q_ref[...], g_ref[...], b_ref[...])
    halves = []
    for h, a in enumerate((a0_ref[...], a1_ref[...])):
        y = jnp.where(a >= 0, a, 0.01 * a)
        halves.append(y * sc[h:h + 1] + off[h:h + 1])
    cat = jnp.concatenate(halves, axis=-1)
    o_ref[...] = jnp.dot(cat, w_ref[...], preferred_element_type=jnp.float32)


def _tc_bn_mm(a_flat, s, q, g2, b2, w_mat):
    nw = w_mat.shape[1]
    small = ((2, 32), lambda i: (0, 0))
    arrs = [
        ((BLK, 32), lambda i: (i, 0)),
        ((BLK, 32), lambda i: (NB + i, 0)),
        small, small, small, small,
        ((C_OUT, nw), lambda i: (0, 0)),
    ]
    return _mm_table_call(nw, _k3_body, arrs)(a_flat, a_flat, s, q, g2, b2,
                                              w_mat)


def _k5_body(c10_ref, c11_ref, c20_ref, c21_ref, s1_ref, q1_ref, s2_ref,
             q2_ref, g1_ref, b1_ref, g2_ref, b2_ref, w_ref, o_ref):
    sc1, off1 = _bn_coeff(s1_ref[...], q1_ref[...], g1_ref[...], b1_ref[...])
    sc2, off2 = _bn_coeff(s2_ref[...], q2_ref[...], g2_ref[...], b2_ref[...])
    halves = []
    for h, (c1, c2) in enumerate(((c10_ref[...], c20_ref[...]),
                                  (c11_ref[...], c21_ref[...]))):
        halves.append(c1 * sc1[h:h + 1] + off1[h:h + 1]
                      + c2 * sc2[h:h + 1] + off2[h:h + 1])
    e = jnp.concatenate(halves, axis=-1)
    o_ref[...] = jnp.dot(e, w_ref[...], preferred_element_type=jnp.float32)


def _tc_bn2_mm(c1_flat, c2_flat, s1, q1, s2, q2, g1h, b1h, g2h, b2h, w_mat):
    nw = w_mat.shape[1]
    small = ((2, 32), lambda i: (0, 0))
    arrs = [
        ((BLK, 32), lambda i: (i, 0)),
        ((BLK, 32), lambda i: (NB + i, 0)),
        ((BLK, 32), lambda i: (i, 0)),
        ((BLK, 32), lambda i: (NB + i, 0)),
        small, small, small, small, small, small, small, small,
        ((C_OUT, nw), lambda i: (0, 0)),
    ]
    return _mm_table_call(nw, _k5_body, arrs)(
        c1_flat, c1_flat, c2_flat, c2_flat, s1, q1, s2, q2,
        g1h, b1h, g2h, b2h, w_mat)


def _k2_body(a0_ref, a1_ref, s_ref, q_ref):
    y0 = a0_ref[...]
    y0 = jnp.where(y0 >= 0, y0, 0.01 * y0)
    y1 = a1_ref[...]
    y1 = jnp.where(y1 >= 0, y1, 0.01 * y1)

    @pl.when(pl.program_id(0) == 0)
    def _():
        s_ref[...] = jnp.zeros_like(s_ref)
        q_ref[...] = jnp.zeros_like(q_ref)

    s_ref[...] += jnp.stack([jnp.sum(y0, 0), jnp.sum(y1, 0)])
    q_ref[...] += jnp.stack([jnp.sum(y0 * y0, 0), jnp.sum(y1 * y1, 0)])


def _tc_stats_leaky(a_flat):
    sspec = pl.BlockSpec((2, 32), lambda i: (0, 0))
    sshape = jax.ShapeDtypeStruct((2, 32), jnp.float32)
    return pl.pallas_call(
        _k2_body,
        grid=(NB,),
        in_specs=[
            pl.BlockSpec((BLK, 32), lambda i: (i, 0)),
            pl.BlockSpec((BLK, 32), lambda i: (NB + i, 0)),
        ],
        out_specs=[sspec, sspec],
        out_shape=[sshape, sshape],
    )(a_flat, a_flat)


def _k4_body(c10_ref, c11_ref, c20_ref, c21_ref, s1_ref, q1_ref, s2_ref,
             q2_ref):
    @pl.when(pl.program_id(0) == 0)
    def _():
        s1_ref[...] = jnp.zeros_like(s1_ref)
        q1_ref[...] = jnp.zeros_like(q1_ref)
        s2_ref[...] = jnp.zeros_like(s2_ref)
        q2_ref[...] = jnp.zeros_like(q2_ref)

    c10 = c10_ref[...]
    c11 = c11_ref[...]
    c20 = c20_ref[...]
    c21 = c21_ref[...]
    s1_ref[...] += jnp.stack([jnp.sum(c10, 0), jnp.sum(c11, 0)])
    q1_ref[...] += jnp.stack([jnp.sum(c10 * c10, 0), jnp.sum(c11 * c11, 0)])
    s2_ref[...] += jnp.stack([jnp.sum(c20, 0), jnp.sum(c21, 0)])
    q2_ref[...] += jnp.stack([jnp.sum(c20 * c20, 0), jnp.sum(c21 * c21, 0)])


def _tc_stats2(c1_flat, c2_flat):
    sspec = pl.BlockSpec((2, 32), lambda i: (0, 0))
    sshape = jax.ShapeDtypeStruct((2, 32), jnp.float32)
    return pl.pallas_call(
        _k4_body,
        grid=(NB,),
        in_specs=[
            pl.BlockSpec((BLK, 32), lambda i: (i, 0)),
            pl.BlockSpec((BLK, 32), lambda i: (NB + i, 0)),
            pl.BlockSpec((BLK, 32), lambda i: (i, 0)),
            pl.BlockSpec((BLK, 32), lambda i: (NB + i, 0)),
        ],
        out_specs=[sspec, sspec, sspec, sspec],
        out_shape=[sshape, sshape, sshape, sshape],
    )(c1_flat, c1_flat, c2_flat, c2_flat)


def _k6_body(a0_ref, a1_ref, o_ref):
    o_ref[...] = jnp.concatenate([a0_ref[...], a1_ref[...]], axis=-1)


def _tc_interleave(a_flat, n_rows):
    nb = n_rows // BLK
    return pl.pallas_call(
        _k6_body,
        grid=(nb,),
        in_specs=[
            pl.BlockSpec((BLK, 32), lambda i: (i, 0)),
            pl.BlockSpec((BLK, 32), lambda i: (nb + i, 0)),
        ],
        out_specs=pl.BlockSpec((BLK, C_OUT), lambda i: (i, 0)),
        out_shape=jax.ShapeDtypeStruct((n_rows, C_OUT), jnp.float32),
    )(a_flat, a_flat)


def _sc_zero_acc(s, zbuf, acc):
    for j in range(_cdiv(ZCHUNKS, NS)):
        ci = s + NS * j

        @pl.when(ci < ZCHUNKS)
        def _():
            pltpu.sync_copy(zbuf, acc.at[pl.ds(ci * CHUNK, CHUNK)])


def _sc_writeback(s, acc, out_hbm, out_base):
    for j in range(_cdiv(WCHUNKS, NS)):
        ci = s + NS * j

        @pl.when(ci < WCHUNKS)
        def _():
            pltpu.sync_copy(acc.at[pl.ds(ci * CHUNK, CHUNK)],
                            out_hbm.at[pl.ds(out_base + ci * CHUNK, CHUNK)])

    @pl.when(s == 0)
    def _():
        pltpu.sync_copy(acc.at[pl.ds(WCHUNKS * CHUNK, WTAIL)],
                        out_hbm.at[pl.ds(out_base + WCHUNKS * CHUNK, WTAIL)])


def _sc_pair_loop(s, tab, g_hbm, s_hbm, goff, soff, iters, acc,
                  idxg, idxs, rows, semg):
    base0 = s * (iters * CHUNK)
    half = iters // 2

    def fire(b, i):
        base = base0 + i * CHUNK
        pltpu.sync_copy(g_hbm.at[pl.ds(goff + base, CHUNK)], idxg[b])
        pltpu.sync_copy(s_hbm.at[pl.ds(soff + base, CHUNK)], idxs[b])
        pltpu.make_async_copy(tab.at[idxg[b]], rows[b], semg[b]).start()

    fire(0, 0)
    fire(1, 1)

    def body(t, _):
        for b in range(2):
            pltpu.make_async_copy(tab.at[idxg[b]], rows[b], semg[b]).wait()
            pltpu.sync_copy(rows[b], acc.at[idxs[b]], add=True)

            @pl.when(t < half - 1)
            def _():
                fire(b, 2 * t + 2 + b)

        return 0

    lax.fori_loop(0, half, body, 0)


def _sc_mesh():
    return plsc.VectorSubcoreMesh(core_axis_name="c", subcore_axis_name="s")


def _sc_scratch():
    return [
        pltpu.VMEM((CHUNK,), jnp.int32),
        pltpu.VMEM((CHUNK,), jnp.int32),
        pltpu.VMEM((CHUNK,), jnp.int32),
        pltpu.VMEM((CHUNK,), jnp.int32),
        pltpu.VMEM((CHUNK, 32), jnp.float32),
        pltpu.VMEM((CHUNK, 32), jnp.float32),
        pltpu.VMEM((CHUNK, 32), jnp.float32),
        pltpu.VMEM_SHARED((ACC_ROWS, 32), jnp.float32),
        pltpu.SemaphoreType.DMA,
        pltpu.SemaphoreType.DMA,
    ]


def _make_sc_conv():
    @functools.partial(
        pl.kernel,
        out_type=jax.ShapeDtypeStruct((2 * N, 32), jnp.float32),
        mesh=_sc_mesh(),
        compiler_params=pltpu.CompilerParams(use_tc_tiling_on_sc=False),
        scratch_types=_sc_scratch(),
    )
    def k(tab, g_hbm, s_hbm, zrows, out_hbm,
          idxg0, idxg1, idxs0, idxs1, rows0, rows1, zbuf, acc, sem0, sem1):
        c = lax.axis_index("c")
        s = lax.axis_index("s")
        pltpu.sync_copy(zrows, zbuf)
        _sc_zero_acc(s, zbuf, acc)
        plsc.subcore_barrier()
        _sc_pair_loop(s, tab, g_hbm, s_hbm, c * PAD27, 0, IT27, acc,
                      (idxg0, idxg1), (idxs0, idxs1), (rows0, rows1),
                      (sem0, sem1))
        plsc.subcore_barrier()
        _sc_writeback(s, acc, out_hbm, c * N)

    return k


def _make_sc_conv2():
    @functools.partial(
        pl.kernel,
        out_type=(jax.ShapeDtypeStruct((2 * N, 32), jnp.float32),
                  jax.ShapeDtypeStruct((2 * N, 32), jnp.float32)),
        mesh=_sc_mesh(),
        compiler_params=pltpu.CompilerParams(use_tc_tiling_on_sc=False),
        scratch_types=_sc_scratch(),
    )
    def k(tab, g1_hbm, s1_hbm, g2_hbm, s2_hbm, zrows, out1, out2,
          idxg0, idxg1, idxs0, idxs1, rows0, rows1, zbuf, acc, sem0, sem1):
        c = lax.axis_index("c")
        s = lax.axis_index("s")
        pltpu.sync_copy(zrows, zbuf)
        for gi, si, oi in ((g1_hbm, s1_hbm, out1), (g2_hbm, s2_hbm, out2)):
            _sc_zero_acc(s, zbuf, acc)
            plsc.subcore_barrier()
            _sc_pair_loop(s, tab, gi, si, c * PAD9, 0, IT9, acc,
                          (idxg0, idxg1), (idxs0, idxs1), (rows0, rows1),
                          (sem0, sem1))
            plsc.subcore_barrier()
            _sc_writeback(s, acc, oi, c * N)
            plsc.subcore_barrier()

    return k


def _make_sc_up():
    @functools.partial(
        pl.kernel,
        out_type=jax.ShapeDtypeStruct((2 * N_OUT, 32), jnp.float32),
        mesh=_sc_mesh(),
        compiler_params=pltpu.CompilerParams(use_tc_tiling_on_sc=False),
        scratch_types=_sc_scratch(),
    )
    def k(tab, g_hbm, s_hbm, zrows, out_hbm,
          idxg0, idxg1, idxs0, idxs1, rows0, rows1, zbuf, acc, sem0, sem1):
        c = lax.axis_index("c")
        s = lax.axis_index("s")
        pltpu.sync_copy(zrows, zbuf)
        for p in range(2):
            _sc_zero_acc(s, zbuf, acc)
            plsc.subcore_barrier()
            _sc_pair_loop(s, tab, g_hbm, s_hbm, c * PAD27, p * PAD27, IT27,
                          acc, (idxg0, idxg1), (idxs0, idxs1), (rows0, rows1),
                          (sem0, sem1))
            plsc.subcore_barrier()
            _sc_writeback(s, acc, out_hbm, c * N_OUT + p * N)
            plsc.subcore_barrier()

    return k


def _prep_w(w, cin):
    kk = w.shape[0]
    return jnp.transpose(w, (1, 0, 2)).reshape(cin, kk * C_OUT)


def _prep_gidx(rb_in, kk2, pad_len, koff=0):
    kk = rb_in.shape[0]
    k_ids = (jnp.arange(kk, dtype=jnp.int32)[:, None] + koff) * 2
    g = rb_in.astype(jnp.int32) * kk2 + k_ids
    g = jnp.pad(g.reshape(-1), (0, pad_len - kk * P))
    return jnp.concatenate([g, g + 1])


def _prep_sidx(rb_out, pad_len):
    s = rb_out.astype(jnp.int32).reshape(-1)
    return jnp.pad(s, (0, pad_len - s.shape[0]), constant_values=DUMMY)


def kernel(x_features, skip_features, W_trans, bn_t_g, bn_t_b, W1, bn1_g,
           bn1_b, W2, bn2_g, bn2_b, W_up, rb_trans_in, rb_trans_out, rb1_in,
           rb1_out, rb2_in, rb2_out, rb_up_in, rb_up_out):
    f32 = jnp.float32

    w_t = _prep_w(W_trans, C_IN)
    w_12 = _prep_w(jnp.concatenate([W1, W2], 0), C_OUT)
    w_up = _prep_w(W_up, C_OUT)

    gt = _prep_gidx(rb_trans_in, 54, PAD27)
    st = _prep_sidx(rb_trans_out, PAD27)
    g1 = _prep_gidx(rb1_in, 36, PAD9)
    s1 = _prep_sidx(rb1_out, PAD9)
    g2 = _prep_gidx(rb2_in, 36, PAD9, 9)
    s2 = _prep_sidx(rb2_out, PAD9)
    gu = _prep_gidx(rb_up_in, 54, PAD27)
    su_raw = rb_up_out.astype(jnp.int32).reshape(-1)
    su_raw = jnp.pad(su_raw, (0, PAD27 - 27 * P), constant_values=1 << 29)
    su0 = jnp.where(su_raw < N, su_raw, DUMMY)
    su1 = jnp.where((su_raw >= N) & (su_raw < 2 * N), su_raw - N, DUMMY)
    su = jnp.concatenate([su0, su1])

    zrows = jnp.zeros((CHUNK, 32), f32)

    g2h_t = bn_t_g.reshape(2, 32).astype(f32)
    b2h_t = bn_t_b.reshape(2, 32).astype(f32)
    g2h_1 = bn1_g.reshape(2, 32).astype(f32)
    b2h_1 = bn1_b.reshape(2, 32).astype(f32)
    g2h_2 = bn2_g.reshape(2, 32).astype(f32)
    b2h_2 = bn2_b.reshape(2, 32).astype(f32)

    tab_t = _tc_mm1(x_features, skip_features, w_t).reshape(54 * N, 32)
    a_flat = _make_sc_conv()(tab_t, gt, st, zrows)

    s_t, q_t = _tc_stats_leaky(a_flat)
    tab_12 = _tc_bn_mm(a_flat, s_t, q_t, g2h_t, b2h_t,
                       w_12).reshape(36 * N, 32)

    c1_flat, c2_flat = _make_sc_conv2()(tab_12, g1, s1, g2, s2, zrows)

    s1s, q1s, s2s, q2s = _tc_stats2(c1_flat, c2_flat)
    tab_up = _tc_bn2_mm(c1_flat, c2_flat, s1s, q1s, s2s, q2s,
                        g2h_1, b2h_1, g2h_2, b2h_2,
                        w_up).reshape(54 * N, 32)

    o_flat = _make_sc_up()(tab_up, gu, su, zrows)
    return _tc_interleave(o_flat, N_OUT)


# speedup vs baseline: 3.2338x; 1.0014x over previous
import functools

import jax
import jax.numpy as jnp
from jax import lax
from jax.experimental import pallas as pl
from jax.experimental.pallas import tpu as pltpu
from jax.experimental.pallas import tpu_sc as plsc

N = 50000
N_OUT = 100000
P = 12500
C_IN = 128
C_OUT = 64

NC = 2
NS = 16
LANES = 16
CHUNK = 128

BLK = 1000
NB = N // BLK
EPS = 1e-5

NP = 50048
NP4 = NP // 4
IT27 = 166
PAD27 = NS * CHUNK * IT27
IT9 = 56
PAD9 = NS * CHUNK * IT9

DUMMY = 50000
ACC_ROWS = 50176
ZCHUNKS = ACC_ROWS // CHUNK
WCHUNKS = N // CHUNK
WTAIL = N - WCHUNKS * CHUNK


def _cdiv(a, b):
    return (a + b - 1) // b


def _mm_table_call(nw, body, arrs):
    return pl.pallas_call(
        body,
        grid=(NB,),
        in_specs=[pl.BlockSpec(s, m) for s, m in arrs],
        out_specs=pl.BlockSpec((BLK, nw), lambda i: (i, 0)),
        out_shape=jax.ShapeDtypeStruct((N, nw), jnp.float32),
    )


def _k1_body(x_ref, w_ref, o_ref):
    o_ref[...] = jnp.dot(x_ref[...], w_ref[0],
                         preferred_element_type=jnp.float32)


def _tc_mm1(xs4, w4):
    nj = w4.shape[0]
    cin4 = w4.shape[1]
    return pl.pallas_call(
        _k1_body,
        grid=(nj,),
        in_specs=[
            pl.BlockSpec((NP4, cin4), lambda j: (0, 0)),
            pl.BlockSpec((1, cin4, 128), lambda j: (j, 0, 0)),
        ],
        out_specs=pl.BlockSpec((NP4, 128), lambda j: (j, 0)),
        out_shape=jax.ShapeDtypeStruct((nj * NP4, 128), jnp.float32),
    )(xs4, w4)


def _bn_coeff(s, q, g, b):
    mu = s * (1.0 / N)
    var = q * (1.0 / N) - mu * mu
    sc = g * lax.rsqrt(var + EPS)
    return sc, b - mu * sc


def _k3_body(a0_ref, a1_ref, s_ref, q_ref, g_ref, b_ref, w_ref, o_ref):
    sc, off = _bn_coeff(s_ref[...], q_ref[...], g_ref[...], b_ref[...])
    halves = []
    for h, a in enumerate((a0_ref[...], a1_ref[...])):
        y = jnp.where(a >= 0, a, 0.01 * a)
        halves.append(y * sc[h:h + 1] + off[h:h + 1])
    cat = jnp.concatenate(halves, axis=-1)
    o_ref[...] = jnp.dot(cat, w_ref[...], preferred_element_type=jnp.float32)


def _tc_bn_mm(a_flat, s, q, g2, b2, w_mat):
    nw = w_mat.shape[1]
    small = ((2, 32), lambda i: (0, 0))
    arrs = [
        ((BLK, 32), lambda i: (i, 0)),
        ((BLK, 32), lambda i: (NB + i, 0)),
        small, small, small, small,
        ((C_OUT, nw), lambda i: (0, 0)),
    ]
    return _mm_table_call(nw, _k3_body, arrs)(a_flat, a_flat, s, q, g2, b2,
                                              w_mat)


def _k5_body(c10_ref, c11_ref, c20_ref, c21_ref, s1_ref, q1_ref, s2_ref,
             q2_ref, g1_ref, b1_ref, g2_ref, b2_ref, w_ref, o_ref):
    sc1, off1 = _bn_coeff(s1_ref[...], q1_ref[...], g1_ref[...], b1_ref[...])
    sc2, off2 = _bn_coeff(s2_ref[...], q2_ref[...], g2_ref[...], b2_ref[...])
    halves = []
    for h, (c1, c2) in enumerate(((c10_ref[...], c20_ref[...]),
                                  (c11_ref[...], c21_ref[...]))):
        halves.append(c1 * sc1[h:h + 1] + off1[h:h + 1]
                      + c2 * sc2[h:h + 1] + off2[h:h + 1])
    e = jnp.concatenate(halves, axis=-1)
    o_ref[...] = jnp.dot(e, w_ref[...], preferred_element_type=jnp.float32)


def _tc_bn2_mm(c1_flat, c2_flat, s1, q1, s2, q2, g1h, b1h, g2h, b2h, w_mat):
    nw = w_mat.shape[1]
    small = ((2, 32), lambda i: (0, 0))
    arrs = [
        ((BLK, 32), lambda i: (i, 0)),
        ((BLK, 32), lambda i: (NB + i, 0)),
        ((BLK, 32), lambda i: (i, 0)),
        ((BLK, 32), lambda i: (NB + i, 0)),
        small, small, small, small, small, small, small, small,
        ((C_OUT, nw), lambda i: (0, 0)),
    ]
    return _mm_table_call(nw, _k5_body, arrs)(
        c1_flat, c1_flat, c2_flat, c2_flat, s1, q1, s2, q2,
        g1h, b1h, g2h, b2h, w_mat)


def _k2_body(a0_ref, a1_ref, s_ref, q_ref):
    y0 = a0_ref[...]
    y0 = jnp.where(y0 >= 0, y0, 0.01 * y0)
    y1 = a1_ref[...]
    y1 = jnp.where(y1 >= 0, y1, 0.01 * y1)

    @pl.when(pl.program_id(0) == 0)
    def _():
        s_ref[...] = jnp.zeros_like(s_ref)
        q_ref[...] = jnp.zeros_like(q_ref)

    s_ref[...] += jnp.stack([jnp.sum(y0, 0), jnp.sum(y1, 0)])
    q_ref[...] += jnp.stack([jnp.sum(y0 * y0, 0), jnp.sum(y1 * y1, 0)])


def _tc_stats_leaky(a_flat):
    sspec = pl.BlockSpec((2, 32), lambda i: (0, 0))
    sshape = jax.ShapeDtypeStruct((2, 32), jnp.float32)
    return pl.pallas_call(
        _k2_body,
        grid=(NB,),
        in_specs=[
            pl.BlockSpec((BLK, 32), lambda i: (i, 0)),
            pl.BlockSpec((BLK, 32), lambda i: (NB + i, 0)),
        ],
        out_specs=[sspec, sspec],
        out_shape=[sshape, sshape],
    )(a_flat, a_flat)


def _k4_body(c10_ref, c11_ref, c20_ref, c21_ref, s1_ref, q1_ref, s2_ref,
             q2_ref):
    @pl.when(pl.program_id(0) == 0)
    def _():
        s1_ref[...] = jnp.zeros_like(s1_ref)
        q1_ref[...] = jnp.zeros_like(q1_ref)
        s2_ref[...] = jnp.zeros_like(s2_ref)
        q2_ref[...] = jnp.zeros_like(q2_ref)

    c10 = c10_ref[...]
    c11 = c11_ref[...]
    c20 = c20_ref[...]
    c21 = c21_ref[...]
    s1_ref[...] += jnp.stack([jnp.sum(c10, 0), jnp.sum(c11, 0)])
    q1_ref[...] += jnp.stack([jnp.sum(c10 * c10, 0), jnp.sum(c11 * c11, 0)])
    s2_ref[...] += jnp.stack([jnp.sum(c20, 0), jnp.sum(c21, 0)])
    q2_ref[...] += jnp.stack([jnp.sum(c20 * c20, 0), jnp.sum(c21 * c21, 0)])


def _tc_stats2(c1_flat, c2_flat):
    sspec = pl.BlockSpec((2, 32), lambda i: (0, 0))
    sshape = jax.ShapeDtypeStruct((2, 32), jnp.float32)
    return pl.pallas_call(
        _k4_body,
        grid=(NB,),
        in_specs=[
            pl.BlockSpec((BLK, 32), lambda i: (i, 0)),
            pl.BlockSpec((BLK, 32), lambda i: (NB + i, 0)),
            pl.BlockSpec((BLK, 32), lambda i: (i, 0)),
            pl.BlockSpec((BLK, 32), lambda i: (NB + i, 0)),
        ],
        out_specs=[sspec, sspec, sspec, sspec],
        out_shape=[sshape, sshape, sshape, sshape],
    )(c1_flat, c1_flat, c2_flat, c2_flat)


def _k6_body(a0_ref, a1_ref, o_ref):
    o_ref[...] = jnp.concatenate([a0_ref[...], a1_ref[...]], axis=-1)


def _tc_interleave(a_flat, n_rows):
    nb = n_rows // BLK
    return pl.pallas_call(
        _k6_body,
        grid=(nb,),
        in_specs=[
            pl.BlockSpec((BLK, 32), lambda i: (i, 0)),
            pl.BlockSpec((BLK, 32), lambda i: (nb + i, 0)),
        ],
        out_specs=pl.BlockSpec((BLK, C_OUT), lambda i: (i, 0)),
        out_shape=jax.ShapeDtypeStruct((n_rows, C_OUT), jnp.float32),
    )(a_flat, a_flat)


def _sc_zero_acc(s, zbuf, acc):
    for j in range(_cdiv(ZCHUNKS, NS)):
        ci = s + NS * j

        @pl.when(ci < ZCHUNKS)
        def _():
            pltpu.sync_copy(zbuf, acc.at[pl.ds(ci * CHUNK, CHUNK)])


def _sc_writeback(s, acc, out_hbm, out_base):
    for j in range(_cdiv(WCHUNKS, NS)):
        ci = s + NS * j

        @pl.when(ci < WCHUNKS)
        def _():
            pltpu.sync_copy(acc.at[pl.ds(ci * CHUNK, CHUNK)],
                            out_hbm.at[pl.ds(out_base + ci * CHUNK, CHUNK)])

    @pl.when(s == 0)
    def _():
        pltpu.sync_copy(acc.at[pl.ds(WCHUNKS * CHUNK, WTAIL)],
                        out_hbm.at[pl.ds(out_base + WCHUNKS * CHUNK, WTAIL)])


def _sc_pair_loop(s, tab, g_hbm, s_hbm, goff, soff, iters, acc,
                  idxg, idxs, rows, semg):
    base0 = s * (iters * CHUNK)
    half = iters // 2

    def fire(b, i):
        base = base0 + i * CHUNK
        pltpu.sync_copy(g_hbm.at[pl.ds(goff + base, CHUNK)], idxg[b])
        pltpu.sync_copy(s_hbm.at[pl.ds(soff + base, CHUNK)], idxs[b])
        pltpu.make_async_copy(tab.at[idxg[b]], rows[b], semg[b]).start()

    fire(0, 0)
    fire(1, 1)

    def body(t, _):
        for b in range(2):
            pltpu.make_async_copy(tab.at[idxg[b]], rows[b], semg[b]).wait()
            pltpu.sync_copy(rows[b], acc.at[idxs[b]], add=True)

            @pl.when(t < half - 1)
            def _():
                fire(b, 2 * t + 2 + b)

        return 0

    lax.fori_loop(0, half, body, 0)


def _sc_mesh():
    return plsc.VectorSubcoreMesh(core_axis_name="c", subcore_axis_name="s")


def _sc_scratch():
    return [
        pltpu.VMEM((CHUNK,), jnp.int32),
        pltpu.VMEM((CHUNK,), jnp.int32),
        pltpu.VMEM((CHUNK,), jnp.int32),
        pltpu.VMEM((CHUNK,), jnp.int32),
        pltpu.VMEM((CHUNK, 32), jnp.float32),
        pltpu.VMEM((CHUNK, 32), jnp.float32),
        pltpu.VMEM((CHUNK, 32), jnp.float32),
        pltpu.VMEM_SHARED((ACC_ROWS, 32), jnp.float32),
        pltpu.SemaphoreType.DMA,
        pltpu.SemaphoreType.DMA,
    ]


def _make_sc_conv():
    @functools.partial(
        pl.kernel,
        out_type=jax.ShapeDtypeStruct((2 * N, 32), jnp.float32),
        mesh=_sc_mesh(),
        compiler_params=pltpu.CompilerParams(use_tc_tiling_on_sc=False),
        scratch_types=_sc_scratch(),
    )
    def k(tab, g_hbm, s_hbm, zrows, out_hbm,
          idxg0, idxg1, idxs0, idxs1, rows0, rows1, zbuf, acc, sem0, sem1):
        c = lax.axis_index("c")
        s = lax.axis_index("s")
        pltpu.sync_copy(zrows, zbuf)
        _sc_zero_acc(s, zbuf, acc)
        plsc.subcore_barrier()
        _sc_pair_loop(s, tab, g_hbm, s_hbm, c * PAD27, 0, IT27, acc,
                      (idxg0, idxg1), (idxs0, idxs1), (rows0, rows1),
                      (sem0, sem1))
        plsc.subcore_barrier()
        _sc_writeback(s, acc, out_hbm, c * N)

    return k


def _make_sc_conv2():
    @functools.partial(
        pl.kernel,
        out_type=(jax.ShapeDtypeStruct((2 * N, 32), jnp.float32),
                  jax.ShapeDtypeStruct((2 * N, 32), jnp.float32)),
        mesh=_sc_mesh(),
        compiler_params=pltpu.CompilerParams(use_tc_tiling_on_sc=False),
        scratch_types=_sc_scratch(),
    )
    def k(tab, g1_hbm, s1_hbm, g2_hbm, s2_hbm, zrows, out1, out2,
          idxg0, idxg1, idxs0, idxs1, rows0, rows1, zbuf, acc, sem0, sem1):
        c = lax.axis_index("c")
        s = lax.axis_index("s")
        pltpu.sync_copy(zrows, zbuf)
        for gi, si, oi in ((g1_hbm, s1_hbm, out1), (g2_hbm, s2_hbm, out2)):
            _sc_zero_acc(s, zbuf, acc)
            plsc.subcore_barrier()
            _sc_pair_loop(s, tab, gi, si, c * PAD9, 0, IT9, acc,
                          (idxg0, idxg1), (idxs0, idxs1), (rows0, rows1),
                          (sem0, sem1))
            plsc.subcore_barrier()
            _sc_writeback(s, acc, oi, c * N)
            plsc.subcore_barrier()

    return k


def _make_sc_up():
    @functools.partial(
        pl.kernel,
        out_type=jax.ShapeDtypeStruct((2 * N_OUT, 32), jnp.float32),
        mesh=_sc_mesh(),
        compiler_params=pltpu.CompilerParams(use_tc_tiling_on_sc=False),
        scratch_types=_sc_scratch(),
    )
    def k(tab, g_hbm, s_hbm, zrows, out_hbm,
          idxg0, idxg1, idxs0, idxs1, rows0, rows1, zbuf, acc, sem0, sem1):
        c = lax.axis_index("c")
        s = lax.axis_index("s")
        pltpu.sync_copy(zrows, zbuf)
        for p in range(2):
            _sc_zero_acc(s, zbuf, acc)
            plsc.subcore_barrier()
            _sc_pair_loop(s, tab, g_hbm, s_hbm, c * PAD27, p * PAD27, IT27,
                          acc, (idxg0, idxg1), (idxs0, idxs1), (rows0, rows1),
                          (sem0, sem1))
            plsc.subcore_barrier()
            _sc_writeback(s, acc, out_hbm, c * N_OUT + p * N)
            plsc.subcore_barrier()

    return k


def _prep_w(w, cin):
    kk = w.shape[0]
    return jnp.transpose(w, (1, 0, 2)).reshape(cin, kk * C_OUT)


def _prep_w4(w, cin):
    kk = w.shape[0]
    wj = jnp.transpose(w.reshape(kk, cin, 2, 32), (0, 2, 1, 3))
    wj = wj.reshape(2 * kk, cin, 32)
    z = jnp.zeros((2 * kk, 4 * cin, 128), jnp.float32)
    for q in range(4):
        z = z.at[:, q * cin:(q + 1) * cin, q * 32:(q + 1) * 32].set(wj)
    return z


def _prep_gidx_j(rb_in, pad_len, koff=0):
    kk = rb_in.shape[0]
    k_ids = (jnp.arange(kk, dtype=jnp.int32)[:, None] + koff) * (2 * NP)
    g = rb_in.astype(jnp.int32) + k_ids
    g = jnp.pad(g.reshape(-1), (0, pad_len - kk * P))
    return jnp.concatenate([g, g + NP])


def _prep_gidx(rb_in, kk2, pad_len, koff=0):
    kk = rb_in.shape[0]
    k_ids = (jnp.arange(kk, dtype=jnp.int32)[:, None] + koff) * 2
    g = rb_in.astype(jnp.int32) * kk2 + k_ids
    g = jnp.pad(g.reshape(-1), (0, pad_len - kk * P))
    return jnp.concatenate([g, g + 1])


def _prep_sidx(rb_out, pad_len):
    s = rb_out.astype(jnp.int32).reshape(-1)
    return jnp.pad(s, (0, pad_len - s.shape[0]), constant_values=DUMMY)


def kernel(x_features, skip_features, W_trans, bn_t_g, bn_t_b, W1, bn1_g,
           bn1_b, W2, bn2_g, bn2_b, W_up, rb_trans_in, rb_trans_out, rb1_in,
           rb1_out, rb2_in, rb2_out, rb_up_in, rb_up_out):
    f32 = jnp.float32

    w4_t = _prep_w4(W_trans, C_IN)
    w_12 = _prep_w(jnp.concatenate([W1, W2], 0), C_OUT)
    w_up = _prep_w(W_up, C_OUT)

    gt = _prep_gidx_j(rb_trans_in, PAD27)
    st = _prep_sidx(rb_trans_out, PAD27)
    g1 = _prep_gidx(rb1_in, 36, PAD9)
    s1 = _prep_sidx(rb1_out, PAD9)
    g2 = _prep_gidx(rb2_in, 36, PAD9, 9)
    s2 = _prep_sidx(rb2_out, PAD9)
    gu = _prep_gidx(rb_up_in, 54, PAD27)
    su_raw = rb_up_out.astype(jnp.int32).reshape(-1)
    su_raw = jnp.pad(su_raw, (0, PAD27 - 27 * P), constant_values=1 << 29)
    su0 = jnp.where(su_raw < N, su_raw, DUMMY)
    su1 = jnp.where((su_raw >= N) & (su_raw < 2 * N), su_raw - N, DUMMY)
    su = jnp.concatenate([su0, su1])

    zrows = jnp.zeros((CHUNK, 32), f32)

    g2h_t = bn_t_g.reshape(2, 32).astype(f32)
    b2h_t = bn_t_b.reshape(2, 32).astype(f32)
    g2h_1 = bn1_g.reshape(2, 32).astype(f32)
    b2h_1 = bn1_b.reshape(2, 32).astype(f32)
    g2h_2 = bn2_g.reshape(2, 32).astype(f32)
    b2h_2 = bn2_b.reshape(2, 32).astype(f32)

    xs4 = jnp.pad(x_features + skip_features, ((0, NP - N), (0, 0)))
    xs4 = xs4.reshape(NP4, 4 * C_IN)
    tab_t = _tc_mm1(xs4, w4_t).reshape(54 * NP, 32)
    a_flat = _make_sc_conv()(tab_t, gt, st, zrows)

    s_t, q_t = _tc_stats_leaky(a_flat)
    tab_12 = _tc_bn_mm(a_flat, s_t, q_t, g2h_t, b2h_t,
                       w_12).reshape(36 * N, 32)

    c1_flat, c2_flat = _make_sc_conv2()(tab_12, g1, s1, g2, s2, zrows)

    s1s, q1s, s2s, q2s = _tc_stats2(c1_flat, c2_flat)
    tab_up = _tc_bn2_mm(c1_flat, c2_flat, s1s, q1s, s2s, q2s,
                        g2h_1, b2h_1, g2h_2, b2h_2,
                        w_up).reshape(54 * N, 32)

    o_flat = _make_sc_up()(tab_up, gu, su, zrows)
    return _tc_interleave(o_flat, N_OUT)
